# baseline (speedup 1.0000x reference)
import functools

import jax
import jax.numpy as jnp
from jax import lax
from jax.experimental import pallas as pl
from jax.experimental.pallas import tpu as pltpu
from jax.experimental.pallas import tpu_sc as plsc

LATENT_DIM = 64
RANK = 16
NUM_STEERS = 2
EPSILON = 0.001
BATCH = 4096

_NC = 2
_NS = 16
_NW = _NC * _NS
_B_PER_W = BATCH // _NW


def _sc_gather_body(users_hbm, items_hbm, uemb_hbm, iemb_hbm,
                    uout_hbm, iout_hbm,
                    uidx_v, iidx_v, urows_v, irows_v, sem_u, sem_i):
    wid = lax.axis_index("s") * _NC + lax.axis_index("c")
    base = wid * _B_PER_W
    pltpu.sync_copy(users_hbm.at[pl.ds(base, _B_PER_W)], uidx_v)
    pltpu.sync_copy(items_hbm.at[pl.ds(base, _B_PER_W)], iidx_v)
    cu = pltpu.async_copy(uemb_hbm.at[uidx_v], urows_v, sem_u)
    ci = pltpu.async_copy(iemb_hbm.at[iidx_v], irows_v, sem_i)
    cu.wait()
    ci.wait()
    pltpu.sync_copy(urows_v, uout_hbm.at[pl.ds(base, _B_PER_W)])
    pltpu.sync_copy(irows_v, iout_hbm.at[pl.ds(base, _B_PER_W)])


@functools.cache
def _sc_gather():
    return pl.kernel(
        _sc_gather_body,
        out_type=(
            jax.ShapeDtypeStruct((BATCH, LATENT_DIM), jnp.float32),
            jax.ShapeDtypeStruct((BATCH, LATENT_DIM), jnp.float32),
        ),
        mesh=plsc.VectorSubcoreMesh(core_axis_name="c", subcore_axis_name="s"),
        compiler_params=pltpu.CompilerParams(use_tc_tiling_on_sc=False),
        scratch_types=[
            pltpu.VMEM((_B_PER_W,), jnp.int32),
            pltpu.VMEM((_B_PER_W,), jnp.int32),
            pltpu.VMEM((_B_PER_W, LATENT_DIM), jnp.float32),
            pltpu.VMEM((_B_PER_W, LATENT_DIM), jnp.float32),
            pltpu.SemaphoreType.DMA,
            pltpu.SemaphoreType.DMA,
        ],
    )


def _tc_score_body(sv_ref, u_ref, v_ref, p1_ref, p2_ref, out_ref):
    a = None
    for s in range(NUM_STEERS):
        p1s = p1_ref[s]
        p2s = p2_ref[s]
        contrib = sv_ref[0, s] * lax.dot_general(
            p1s, p2s, (((1,), (1,)), ((), ())),
            preferred_element_type=jnp.float32)
        a = contrib if a is None else a + contrib
    u = u_ref[...]
    v = v_ref[...]
    t = lax.dot_general(v, a, (((1,), (0,)), ((), ())),
                        preferred_element_type=jnp.float32)
    out_ref[...] = jnp.sum(u * (v + EPSILON * t), axis=1)


_tc_score = pl.pallas_call(
    _tc_score_body,
    out_shape=jax.ShapeDtypeStruct((BATCH,), jnp.float32),
    in_specs=[
        pl.BlockSpec(memory_space=pltpu.SMEM),
        pl.BlockSpec(memory_space=pltpu.VMEM),
        pl.BlockSpec(memory_space=pltpu.VMEM),
        pl.BlockSpec(memory_space=pltpu.VMEM),
        pl.BlockSpec(memory_space=pltpu.VMEM),
    ],
)


def kernel(users, items, user_emb, item_emb, projector1, projector2,
           steer_values):
    users = users.astype(jnp.int32)
    items = items.astype(jnp.int32)
    u_rows, v_rows = _sc_gather()(users, items, user_emb, item_emb)
    return _tc_score(steer_values, u_rows, v_rows, projector1, projector2)
